# Initial kernel scaffold; baseline (speedup 1.0000x reference)
import jax
import jax.numpy as jnp
from jax.experimental import pallas as pl


def kernel(x, edge_index, batch, W1, b1, W2, b2, W_ih, W_hh, b_ih, b_hh, Wfc, bfc):
    raise NotImplementedError("write your pallas kernel here")


# speedup vs baseline: 3.6065x; 3.6065x over previous
import functools

import jax
import jax.numpy as jnp
from jax import lax
from jax.experimental import pallas as pl
from jax.experimental.pallas import tpu as pltpu
from jax.experimental.pallas import tpu_sc as plsc

N = 10000
E = 160000
HALF = 5000
CH = 128
CPT = 80
NCHUNK = 16 * CPT
EPAD = NCHUNK * CH
ROWS_ACC = 5120
DUMMY0 = 5008
STRIPE = ROWS_ACC // 16


def _z_body(x_ref, w_ref, dst_ref, xw_ref, dloc_ref):
    i = pl.program_id(0)
    xw_ref[...] = jnp.dot(x_ref[...], w_ref[...],
                          preferred_element_type=jnp.float32)

    @pl.when(i == 0)
    def _():
        d = dst_ref[...]
        dummy = DUMMY0 + (d & 63)
        dloc0 = jnp.where((d >= 0) & (d < HALF), d, dummy)
        dloc1 = jnp.where(d >= HALF, d - HALF, dummy)
        dloc_ref[...] = jnp.stack([dloc0, dloc1])


def _tc_z(x, W1, dst2d):
    return pl.pallas_call(
        _z_body,
        grid=(10,),
        in_specs=[
            pl.BlockSpec((1000, 256), lambda i: (i, 0)),
            pl.BlockSpec((256, 256), lambda i: (0, 0)),
            pl.BlockSpec((NCHUNK, CH), lambda i: (0, 0)),
        ],
        out_specs=[
            pl.BlockSpec((1000, 256), lambda i: (i, 0)),
            pl.BlockSpec((2, NCHUNK, CH), lambda i: (0, 0, 0)),
        ],
        out_shape=[
            jax.ShapeDtypeStruct((N, 256), jnp.float32),
            jax.ShapeDtypeStruct((2, NCHUNK, CH), jnp.int32),
        ],
    )(x, W1, dst2d)


def _deg_body(dloc_hbm, ones_hbm, zvec_hbm, deg_hbm, dlocb, ones_v, acc):
    c = lax.axis_index("c")
    s = lax.axis_index("s")
    pltpu.sync_copy(zvec_hbm, acc.at[pl.ds(s * STRIPE, STRIPE)])
    pltpu.sync_copy(ones_hbm, ones_v)
    pltpu.sync_copy(dloc_hbm.at[c, pl.ds(s * CPT, CPT)], dlocb)
    plsc.subcore_barrier()

    def chunk(k, carry):
        pltpu.sync_copy(ones_v, acc.at[dlocb.at[k]], add=True)
        return carry

    lax.fori_loop(0, CPT, chunk, 0)
    plsc.subcore_barrier()

    @pl.when(s < 15)
    def _():
        pltpu.sync_copy(acc.at[pl.ds(s * STRIPE, STRIPE)],
                        deg_hbm.at[pl.ds(c * HALF + s * STRIPE, STRIPE)])

    @pl.when(s == 15)
    def _():
        pltpu.sync_copy(acc.at[pl.ds(15 * STRIPE, HALF - 15 * STRIPE)],
                        deg_hbm.at[pl.ds(c * HALF + 15 * STRIPE,
                                         HALF - 15 * STRIPE)])


def _sc_deg(dloc2d, ones128, zrows):
    kfn = functools.partial(
        pl.kernel,
        out_type=jax.ShapeDtypeStruct((N, 128), jnp.float32),
        mesh=plsc.VectorSubcoreMesh(core_axis_name="c", subcore_axis_name="s"),
        scratch_types=[
            pltpu.VMEM((CPT, CH), jnp.int32),
            pltpu.VMEM((CH, 128), jnp.float32),
            pltpu.VMEM_SHARED((ROWS_ACC, 128), jnp.float32),
        ],
    )
    return kfn(_deg_body)(dloc2d, ones128, zrows)


def _agg_body(src_hbm, dloc_hbm, zrows_hbm, y_hbm, out_hbm,
              srcb, dlocb, rows0, rows1, sem0, sem1, acc):
    c = lax.axis_index("c")
    s = lax.axis_index("s")
    pltpu.sync_copy(zrows_hbm, acc.at[pl.ds(s * STRIPE, STRIPE)])
    pltpu.sync_copy(src_hbm.at[pl.ds(s * CPT, CPT)], srcb)
    pltpu.sync_copy(dloc_hbm.at[c, pl.ds(s * CPT, CPT)], dlocb)
    plsc.subcore_barrier()

    def start(k, buf, sem):
        pltpu.make_async_copy(y_hbm.at[srcb.at[k]], buf, sem).start()

    def wait(buf, sem):
        pltpu.make_async_copy(y_hbm.at[srcb.at[0]], buf, sem).wait()

    start(0, rows0, sem0)

    def pair(k2, carry):
        k0 = 2 * k2
        k1 = 2 * k2 + 1

        @pl.when(k1 < CPT)
        def _():
            start(k1, rows1, sem1)

        wait(rows0, sem0)
        pltpu.sync_copy(rows0, acc.at[dlocb.at[k0]], add=True)

        @pl.when(k0 + 2 < CPT)
        def _():
            start(k0 + 2, rows0, sem0)

        @pl.when(k1 < CPT)
        def _():
            wait(rows1, sem1)
            pltpu.sync_copy(rows1, acc.at[dlocb.at[k1]], add=True)

        return carry

    lax.fori_loop(0, (CPT + 1) // 2, pair, 0)
    plsc.subcore_barrier()

    @pl.when(s < 15)
    def _():
        pltpu.sync_copy(acc.at[pl.ds(s * STRIPE, STRIPE)],
                        out_hbm.at[pl.ds(c * HALF + s * STRIPE, STRIPE)])

    @pl.when(s == 15)
    def _():
        pltpu.sync_copy(acc.at[pl.ds(15 * STRIPE, HALF - 15 * STRIPE)],
                        out_hbm.at[pl.ds(c * HALF + 15 * STRIPE,
                                         HALF - 15 * STRIPE)])


def _sc_agg(src2d, dloc2d, zrows, y_half):
    kfn = functools.partial(
        pl.kernel,
        out_type=jax.ShapeDtypeStruct((N, 128), jnp.float32),
        mesh=plsc.VectorSubcoreMesh(core_axis_name="c", subcore_axis_name="s"),
        scratch_types=[
            pltpu.VMEM((CPT, CH), jnp.int32),
            pltpu.VMEM((CPT, CH), jnp.int32),
            pltpu.VMEM((CH, 128), jnp.float32),
            pltpu.VMEM((CH, 128), jnp.float32),
            pltpu.SemaphoreType.DMA,
            pltpu.SemaphoreType.DMA,
            pltpu.VMEM_SHARED((ROWS_ACC, 128), jnp.float32),
        ],
    )
    return kfn(_agg_body)(src2d, dloc2d, zrows, y_half)


def _b_body(xw_ref, deg_ref, ya_ref, yb_ref, dinv_ref):
    dinv = lax.rsqrt(deg_ref[...][:, :1] + 1.0)
    y = xw_ref[...] * dinv
    ya_ref[...] = y[:, 0:128]
    yb_ref[...] = y[:, 128:256]
    dinv_ref[...] = dinv


def _tc_b(xw1, deg):
    return pl.pallas_call(
        _b_body,
        grid=(10,),
        in_specs=[
            pl.BlockSpec((1000, 256), lambda i: (i, 0)),
            pl.BlockSpec((1000, 128), lambda i: (i, 0)),
        ],
        out_specs=[
            pl.BlockSpec((1000, 128), lambda i: (i, 0)),
            pl.BlockSpec((1000, 128), lambda i: (i, 0)),
            pl.BlockSpec((1000, 1), lambda i: (i, 0)),
        ],
        out_shape=[
            jax.ShapeDtypeStruct((N, 128), jnp.float32),
            jax.ShapeDtypeStruct((N, 128), jnp.float32),
            jax.ShapeDtypeStruct((N, 1), jnp.float32),
        ],
    )(xw1, deg)


def _d_body(agga_ref, aggb_ref, ya_ref, yb_ref, dinv_ref, b_ref, w_ref,
            y2a_ref, y2b_ref):
    dinv = dinv_ref[...]
    agg = jnp.concatenate([agga_ref[...], aggb_ref[...]], axis=1)
    y1 = jnp.concatenate([ya_ref[...], yb_ref[...]], axis=1)
    h1 = jnp.maximum(dinv * (agg + y1) + b_ref[...], 0.0)
    y2 = jnp.dot(h1, w_ref[...], preferred_element_type=jnp.float32) * dinv
    y2a_ref[...] = y2[:, 0:128]
    y2b_ref[...] = y2[:, 128:256]


def _tc_d(agg1a, agg1b, y1a, y1b, dinv, b1, W2):
    return pl.pallas_call(
        _d_body,
        grid=(10,),
        in_specs=[
            pl.BlockSpec((1000, 128), lambda i: (i, 0)),
            pl.BlockSpec((1000, 128), lambda i: (i, 0)),
            pl.BlockSpec((1000, 128), lambda i: (i, 0)),
            pl.BlockSpec((1000, 128), lambda i: (i, 0)),
            pl.BlockSpec((1000, 1), lambda i: (i, 0)),
            pl.BlockSpec((1, 256), lambda i: (0, 0)),
            pl.BlockSpec((256, 256), lambda i: (0, 0)),
        ],
        out_specs=[
            pl.BlockSpec((1000, 128), lambda i: (i, 0)),
            pl.BlockSpec((1000, 128), lambda i: (i, 0)),
        ],
        out_shape=[
            jax.ShapeDtypeStruct((N, 128), jnp.float32),
            jax.ShapeDtypeStruct((N, 128), jnp.float32),
        ],
    )(agg1a, agg1b, y1a, y1b, dinv, b1, W2)


def _f_body(agga_ref, aggb_ref, ya_ref, yb_ref, dinv_ref, b2_ref, batch_ref,
            wih_ref, bih_ref, bhh_ref, wfc_ref, bfc_ref,
            out_ref, sum_acc, cnt_acc):
    i = pl.program_id(0)

    @pl.when(i == 0)
    def _():
        sum_acc[...] = jnp.zeros_like(sum_acc)
        cnt_acc[...] = jnp.zeros_like(cnt_acc)

    dinv = dinv_ref[...]
    agg = jnp.concatenate([agga_ref[...], aggb_ref[...]], axis=1)
    y2 = jnp.concatenate([ya_ref[...], yb_ref[...]], axis=1)
    h2 = jnp.maximum(dinv * (agg + y2) + b2_ref[...], 0.0)
    b = batch_ref[...]
    gids = lax.broadcasted_iota(jnp.int32, (1000, 64), 1)
    onehot = (b == gids).astype(jnp.float32)
    sum_acc[...] += lax.dot_general(onehot, h2, (((0,), (0,)), ((), ())),
                                    preferred_element_type=jnp.float32)
    cnt_acc[...] += lax.dot_general(onehot, jnp.ones((1000, 1), jnp.float32),
                                    (((0,), (0,)), ((), ())),
                                    preferred_element_type=jnp.float32)

    @pl.when(i == 9)
    def _():
        cnt = jnp.maximum(cnt_acc[...][:, :1], 1.0)
        pooled = sum_acc[...] / cnt
        gi = lax.dot_general(pooled, wih_ref[...], (((1,), (1,)), ((), ())),
                             preferred_element_type=jnp.float32) + bih_ref[...]
        bhh = bhh_ref[...]
        r = jax.nn.sigmoid(gi[:, 0:256] + bhh[:, 0:256])
        z = jax.nn.sigmoid(gi[:, 256:512] + bhh[:, 256:512])
        n = jnp.tanh(gi[:, 512:768] + r * bhh[:, 512:768])
        gru = (1.0 - z) * n
        out_ref[...] = lax.dot_general(gru, wfc_ref[...],
                                       (((1,), (1,)), ((), ())),
                                       preferred_element_type=jnp.float32
                                       ) + bfc_ref[...]


def _tc_f(agg2a, agg2b, y2a, y2b, dinv, b2, batch2d,
          W_ih, b_ih2d, b_hh2d, Wfc, bfc2d):
    return pl.pallas_call(
        _f_body,
        grid=(10,),
        in_specs=[
            pl.BlockSpec((1000, 128), lambda i: (i, 0)),
            pl.BlockSpec((1000, 128), lambda i: (i, 0)),
            pl.BlockSpec((1000, 128), lambda i: (i, 0)),
            pl.BlockSpec((1000, 128), lambda i: (i, 0)),
            pl.BlockSpec((1000, 1), lambda i: (i, 0)),
            pl.BlockSpec((1, 256), lambda i: (0, 0)),
            pl.BlockSpec((1000, 1), lambda i: (i, 0)),
            pl.BlockSpec((768, 256), lambda i: (0, 0)),
            pl.BlockSpec((1, 768), lambda i: (0, 0)),
            pl.BlockSpec((1, 768), lambda i: (0, 0)),
            pl.BlockSpec((10, 256), lambda i: (0, 0)),
            pl.BlockSpec((1, 10), lambda i: (0, 0)),
        ],
        out_specs=pl.BlockSpec((64, 10), lambda i: (0, 0)),
        out_shape=jax.ShapeDtypeStruct((64, 10), jnp.float32),
        scratch_shapes=[
            pltpu.VMEM((64, 256), jnp.float32),
            pltpu.VMEM((64, 1), jnp.float32),
        ],
    )(agg2a, agg2b, y2a, y2b, dinv, b2, batch2d,
      W_ih, b_ih2d, b_hh2d, Wfc, bfc2d)


@jax.jit
def kernel(x, edge_index, batch, W1, b1, W2, b2, W_ih, W_hh, b_ih, b_hh,
           Wfc, bfc):
    src = edge_index[0]
    dst = edge_index[1]
    pad = EPAD - E
    src2d = jnp.concatenate(
        [src, jnp.zeros((pad,), jnp.int32)]).reshape(NCHUNK, CH)
    dst2d = jnp.concatenate(
        [dst, jnp.full((pad,), -1, jnp.int32)]).reshape(NCHUNK, CH)

    xw1, dloc2d = _tc_z(x, W1, dst2d)

    ones128 = jnp.ones((CH, 128), jnp.float32)
    zrows = jnp.zeros((STRIPE, 128), jnp.float32)

    deg = _sc_deg(dloc2d, ones128, zrows)
    y1a, y1b, dinv = _tc_b(xw1, deg)
    agg1a = _sc_agg(src2d, dloc2d, zrows, y1a)
    agg1b = _sc_agg(src2d, dloc2d, zrows, y1b)
    y2a, y2b = _tc_d(agg1a, agg1b, y1a, y1b, dinv, b1.reshape(1, 256), W2)
    agg2a = _sc_agg(src2d, dloc2d, zrows, y2a)
    agg2b = _sc_agg(src2d, dloc2d, zrows, y2b)
    out = _tc_f(agg2a, agg2b, y2a, y2b, dinv, b2.reshape(1, 256),
                batch.reshape(N, 1),
                W_ih, b_ih.reshape(1, 768), b_hh.reshape(1, 768),
                Wfc, bfc.reshape(1, 10))
    return out
